# baseline (speedup 1.0000x reference)
import functools

import jax
import jax.numpy as jnp
from jax import lax
from jax.experimental import pallas as pl
from jax.experimental.pallas import tpu as pltpu
from jax.experimental.pallas import tpu_sc as plsc

MAX_NEIGHBORS = 10
PAD_VALUE = -1
EMBED_D = 128

_NC = 2
_NS = 16
_NW = _NC * _NS


def _tc_project(tbl, W, blk):
  N, D = tbl.shape
  E = W.shape[1]
  assert N % blk == 0

  def body(t_r, w_r, o_r):
    o_r[...] = lax.dot_general(t_r[...], w_r[...], (((1,), (0,)), ((), ())),
                               preferred_element_type=jnp.float32)

  return pl.pallas_call(
      body,
      grid=(N // blk,),
      in_specs=[
          pl.BlockSpec((blk, D), lambda i: (i, 0)),
          pl.BlockSpec((D, E), lambda i: (0, 0)),
      ],
      out_specs=pl.BlockSpec((blk, E), lambda i: (i, 0)),
      out_shape=jax.ShapeDtypeStruct((N, E), jnp.float32),
  )(tbl, W)


def _sc_gene_sums(idx_g, w_g, gene_features, B):
  D = EMBED_D
  epw = B // _NW
  ipw = epw * MAX_NEIGHBORS

  mesh = plsc.VectorSubcoreMesh(core_axis_name="c", subcore_axis_name="s")

  @functools.partial(
      pl.kernel,
      out_type=jax.ShapeDtypeStruct((B, D), jnp.float32),
      mesh=mesh,
      scratch_types=[
          pltpu.VMEM((ipw,), jnp.int32),
          pltpu.VMEM((epw * 16,), jnp.float32),
          pltpu.VMEM((ipw, D), jnp.float32),
          pltpu.VMEM((epw, D), jnp.float32),
          pltpu.SemaphoreType.DMA,
      ],
  )
  def k(idx_h, w_h, gene_h, out_h, idx_v, w_v, rows_v, out_v, sem):
    wid = lax.axis_index("s") * _NC + lax.axis_index("c")
    base_e = wid * epw
    pltpu.sync_copy(idx_h.at[pl.ds(wid * ipw, ipw)], idx_v)
    pltpu.sync_copy(w_h.at[pl.ds(wid * epw * 16, epw * 16)], w_v)
    for c0, csz in ((0, 128), (128, 128), (256, 64)):
      pltpu.make_async_copy(
          gene_h.at[idx_v.at[pl.ds(c0, csz)]],
          rows_v.at[pl.ds(c0, csz)], sem).start()
    pltpu.make_async_copy(gene_h.at[pl.ds(0, ipw)], rows_v, sem).wait()

    def elem(e, carry):
      wvec = w_v[pl.ds(pl.multiple_of(e * 16, 16), 16)]
      ws = [wvec[j] for j in range(MAX_NEIGHBORS)]
      row0 = e * MAX_NEIGHBORS
      for c in range(D // 16):
        sl = pl.ds(c * 16, 16)
        acc0 = rows_v[row0 + 0, sl] * ws[0]
        acc1 = rows_v[row0 + 1, sl] * ws[1]
        for j in range(2, MAX_NEIGHBORS, 2):
          acc0 = acc0 + rows_v[row0 + j, sl] * ws[j]
          acc1 = acc1 + rows_v[row0 + j + 1, sl] * ws[j + 1]
        out_v[e, sl] = acc0 + acc1
      return carry

    lax.fori_loop(0, epw, elem, 0)
    pltpu.sync_copy(out_v, out_h.at[pl.ds(base_e, epw)])

  return k(idx_g, w_g, gene_features)


def _sc_gather_sums(idx_all, w_all, p_cell, p_drug, B):
  D = EMBED_D
  assert B % _NW == 0
  epw = B // _NW
  ipw = epw * MAX_NEIGHBORS
  iall = 2 * ipw + epw
  wall = 2 * epw * 16

  mesh = plsc.VectorSubcoreMesh(core_axis_name="c", subcore_axis_name="s")

  @functools.partial(
      pl.kernel,
      out_type=[jax.ShapeDtypeStruct((B, D), jnp.float32)] * 3,
      mesh=mesh,
      scratch_types=[
          pltpu.VMEM((iall,), jnp.int32),
          pltpu.VMEM((wall,), jnp.float32),
          pltpu.VMEM((ipw, D), jnp.float32),
          pltpu.VMEM((ipw, D), jnp.float32),
          pltpu.VMEM((epw, D), jnp.float32),
          pltpu.VMEM((epw, D), jnp.float32),
          pltpu.VMEM((epw, D), jnp.float32),
          pltpu.SemaphoreType.DMA,
          pltpu.SemaphoreType.DMA,
          pltpu.SemaphoreType.DMA,
          pltpu.SemaphoreType.DMA,
      ],
  )
  def k(idx_all_h, w_all_h, pc_h, pd_h,
        sums_c_h, sums_d_h, self_h,
        idx_v, w_v, rows_a, rows_b, out_c, out_d, self_v,
        sem_a, sem_b, sem_s, sem_f):
    wid = lax.axis_index("s") * _NC + lax.axis_index("c")
    base_e = wid * epw
    rows_bufs = (rows_a, rows_b)
    gather_sems = (sem_a, sem_b)

    pltpu.sync_copy(idx_all_h.at[pl.ds(wid * iall, iall)], idx_v)
    pltpu.sync_copy(w_all_h.at[pl.ds(wid * wall, wall)], w_v)

    def fire_gathers(t, tbl_h, q):
      i0 = t * ipw
      for c0, csz in ((0, 128), (128, 128), (256, 64)):
        pltpu.make_async_copy(
            tbl_h.at[idx_v.at[pl.ds(i0 + c0, csz)]],
            rows_bufs[q].at[pl.ds(c0, csz)], gather_sems[q]).start()

    def drain_gathers(q, tbl_h):
      pltpu.make_async_copy(tbl_h.at[pl.ds(0, ipw)], rows_bufs[q],
                            gather_sems[q]).wait()

    def accum_type(t, q, out_v):
      rows_v = rows_bufs[q]
      w0 = t * epw * 16

      def elem(e, carry):
        wvec = w_v[pl.ds(pl.multiple_of(w0 + e * 16, 16), 16)]
        ws = [wvec[j] for j in range(MAX_NEIGHBORS)]
        row0 = e * MAX_NEIGHBORS
        for c in range(D // 16):
          sl = pl.ds(c * 16, 16)
          acc0 = rows_v[row0 + 0, sl] * ws[0]
          acc1 = rows_v[row0 + 1, sl] * ws[1]
          for j in range(2, MAX_NEIGHBORS, 2):
            acc0 = acc0 + rows_v[row0 + j, sl] * ws[j]
            acc1 = acc1 + rows_v[row0 + j + 1, sl] * ws[j + 1]
          out_v[e, sl] = acc0 + acc1
        return carry

      lax.fori_loop(0, epw, elem, 0)

    def store(out_v, out_h):
      pltpu.make_async_copy(out_v, out_h.at[pl.ds(base_e, epw)],
                            sem_s).start()

    tables = (pc_h, pd_h)
    outs_v = (out_c, out_d)
    outs_h = (sums_c_h, sums_d_h)

    fire_gathers(0, tables[0], 0)
    fire_gathers(1, tables[1], 1)
    pltpu.make_async_copy(
        pd_h.at[idx_v.at[pl.ds(2 * ipw, epw)]], self_v, sem_f).start()
    for t in range(2):
      q = t % 2
      drain_gathers(q, tables[t])
      accum_type(t, q, outs_v[t])
      store(outs_v[t], outs_h[t])

    pltpu.make_async_copy(pd_h.at[pl.ds(0, epw)], self_v, sem_f).wait()
    store(self_v, self_h)

    pltpu.make_async_copy(sums_c_h.at[pl.ds(0, 3 * epw)],
                          rows_a.at[pl.ds(0, 3 * epw)], sem_s).wait()

  return k(idx_all, w_all, p_cell, p_drug)


def _tc_mlp(self_p, sums_c, sums_d, sums_g, cnt_c, cnt_d, cnt_g,
            b_cell, b_drug, W_gene, b_gene, W_l1, b_l1, W_l2, b_l2):
  B = self_p.shape[0]
  f32 = jnp.float32
  inv_m = 1.0 / MAX_NEIGHBORS

  def dot(a, b):
    return lax.dot_general(a, b, (((1,), (0,)), ((), ())),
                           preferred_element_type=f32)

  def body(sp_r, sc_r, sd_r, sg_r, cc_r, cd_r, cg_r,
           bc_r, bd_r, Wg_r, bg_r, Wl1_r, bl1_r, Wl2_r, bl2_r, out_r):
    h = sp_r[...] + bd_r[...]
    agg_c = (sc_r[...] + cc_r[...] * bc_r[...]) * inv_m
    agg_d = (sd_r[...] + cd_r[...] * bd_r[...]) * inv_m
    agg_g = (dot(sg_r[...], Wg_r[...]) + cg_r[...] * bg_r[...]) * inv_m
    for Wl_r, bl_r in ((Wl1_r, bl1_r), (Wl2_r, bl2_r)):
      Wl = Wl_r[...]
      pre = (dot(h, Wl[0:EMBED_D]) + dot(agg_c, Wl[EMBED_D:2 * EMBED_D])
             + dot(agg_d, Wl[2 * EMBED_D:3 * EMBED_D])
             + dot(agg_g, Wl[3 * EMBED_D:4 * EMBED_D]) + bl_r[...])
      h = jnp.maximum(pre, 0.0)
    out_r[...] = h

  E = EMBED_D
  full = lambda shape: pl.BlockSpec(shape, lambda: tuple(0 for _ in shape))
  b2 = lambda v: v.reshape(1, E)
  c2 = lambda v: v.reshape(B, 1)
  return pl.pallas_call(
      body,
      in_specs=[
          full((B, E)), full((B, E)), full((B, E)), full((B, E)),
          full((B, 1)), full((B, 1)), full((B, 1)),
          full((1, E)), full((1, E)),
          full((E, E)), full((1, E)),
          full((4 * E, E)), full((1, E)),
          full((4 * E, E)), full((1, E)),
      ],
      out_specs=full((B, E)),
      out_shape=jax.ShapeDtypeStruct((B, E), jnp.float32),
  )(self_p, sums_c, sums_d, sums_g, c2(cnt_c), c2(cnt_d), c2(cnt_g),
    b2(b_cell), b2(b_drug), W_gene, b2(b_gene),
    W_l1, b2(b_l1), W_l2, b2(b_l2))


def kernel(id_batch, neigh_cell, neigh_drug, neigh_gene,
           cell_features, drug_features, gene_features,
           W_cell, b_cell, W_drug, b_drug, W_gene, b_gene,
           W_l1, b_l1, W_l2, b_l2):
  def prep(neigh):
    mask = neigh != PAD_VALUE
    safe = jnp.where(mask, neigh, 0).astype(jnp.int32)
    idx = safe.reshape(-1)
    w = jnp.pad(mask.astype(jnp.float32),
                ((0, 0), (0, 16 - MAX_NEIGHBORS))).reshape(-1)
    cnt = mask.sum(axis=1).astype(jnp.float32)
    return idx, w, cnt

  idx_c, w_c, cnt_c = prep(neigh_cell)
  idx_d, w_d, cnt_d = prep(neigh_drug)
  idx_g, w_g, cnt_g = prep(neigh_gene)

  B = id_batch.shape[0]
  epw = B // _NW
  ipw = epw * MAX_NEIGHBORS
  r = lambda a, n: a.reshape(_NW, n)
  idx_all = jnp.concatenate(
      [r(idx_c, ipw), r(idx_d, ipw),
       r(id_batch.astype(jnp.int32), epw)], axis=1).reshape(-1)
  w_all = jnp.concatenate(
      [r(w_c, epw * 16), r(w_d, epw * 16)], axis=1).reshape(-1)

  sums_g = _sc_gene_sums(idx_g, w_g, gene_features, B)

  p_drug = _tc_project(drug_features, W_drug, 400)
  p_cell = _tc_project(cell_features, W_cell, 400)

  sums_c, sums_d, self_p = _sc_gather_sums(
      idx_all, w_all, p_cell, p_drug, B)

  return _tc_mlp(self_p, sums_c, sums_d, sums_g, cnt_c, cnt_d, cnt_g,
                 b_cell, b_drug, W_gene, b_gene, W_l1, b_l1, W_l2, b_l2)


# speedup vs baseline: 1.1942x; 1.1942x over previous
import functools

import jax
import jax.numpy as jnp
from jax import lax
from jax.experimental import pallas as pl
from jax.experimental.pallas import tpu as pltpu
from jax.experimental.pallas import tpu_sc as plsc

MAX_NEIGHBORS = 10
PAD_VALUE = -1
EMBED_D = 128

_NC = 2
_NS = 16
_NW = _NC * _NS


def _tc_project(tbl, W, blk):
  N, D = tbl.shape
  E = W.shape[1]
  assert N % blk == 0

  def body(t_r, w_r, o_r):
    o_r[...] = lax.dot_general(t_r[...], w_r[...], (((1,), (0,)), ((), ())),
                               preferred_element_type=jnp.float32)

  return pl.pallas_call(
      body,
      grid=(N // blk,),
      in_specs=[
          pl.BlockSpec((blk, D), lambda i: (i, 0)),
          pl.BlockSpec((D, E), lambda i: (0, 0)),
      ],
      out_specs=pl.BlockSpec((blk, E), lambda i: (i, 0)),
      out_shape=jax.ShapeDtypeStruct((N, E), jnp.float32),
  )(tbl, W)


def _sc_gather_sums(idx_all, w_all, p_cell, p_drug, gene_features, B):
  D = EMBED_D
  assert B % _NW == 0
  epw = B // _NW
  ipw = epw * MAX_NEIGHBORS
  iall = 3 * ipw + epw
  wall = 3 * epw * 16

  mesh = plsc.VectorSubcoreMesh(core_axis_name="c", subcore_axis_name="s")

  @functools.partial(
      pl.kernel,
      out_type=[jax.ShapeDtypeStruct((B, D), jnp.float32)] * 4,
      mesh=mesh,
      scratch_types=[
          pltpu.VMEM((iall,), jnp.int32),
          pltpu.VMEM((wall,), jnp.float32),
          pltpu.VMEM((ipw, D), jnp.float32),
          pltpu.VMEM((ipw, D), jnp.float32),
          pltpu.VMEM((epw, D), jnp.float32),
          pltpu.VMEM((epw, D), jnp.float32),
          pltpu.VMEM((epw, D), jnp.float32),
          pltpu.VMEM((epw, D), jnp.float32),
          pltpu.SemaphoreType.DMA,
          pltpu.SemaphoreType.DMA,
          pltpu.SemaphoreType.DMA,
          pltpu.SemaphoreType.DMA,
      ],
  )
  def k(idx_all_h, w_all_h, pc_h, pd_h, gene_h,
        sums_c_h, sums_d_h, sums_g_h, self_h,
        idx_v, w_v, rows_a, rows_b, out_c, out_d, out_g, self_v,
        sem_a, sem_b, sem_s, sem_f):
    wid = lax.axis_index("s") * _NC + lax.axis_index("c")
    base_e = wid * epw
    rows_bufs = (rows_a, rows_b)
    gather_sems = (sem_a, sem_b)

    pltpu.sync_copy(idx_all_h.at[pl.ds(wid * iall, iall)], idx_v)
    pltpu.sync_copy(w_all_h.at[pl.ds(wid * wall, wall)], w_v)

    def fire_gathers(t, tbl_h, q):
      i0 = t * ipw
      for c0, csz in ((0, 128), (128, 128), (256, 64)):
        pltpu.make_async_copy(
            tbl_h.at[idx_v.at[pl.ds(i0 + c0, csz)]],
            rows_bufs[q].at[pl.ds(c0, csz)], gather_sems[q]).start()

    def drain_gathers(q, tbl_h):
      pltpu.make_async_copy(tbl_h.at[pl.ds(0, ipw)], rows_bufs[q],
                            gather_sems[q]).wait()

    def accum_type(t, q, out_v):
      rows_v = rows_bufs[q]
      w0 = t * epw * 16

      def elem(e, carry):
        wvec = w_v[pl.ds(pl.multiple_of(w0 + e * 16, 16), 16)]
        ws = [wvec[j] for j in range(MAX_NEIGHBORS)]
        row0 = e * MAX_NEIGHBORS
        for c in range(D // 16):
          sl = pl.ds(c * 16, 16)
          acc0 = rows_v[row0 + 0, sl] * ws[0]
          acc1 = rows_v[row0 + 1, sl] * ws[1]
          for j in range(2, MAX_NEIGHBORS, 2):
            acc0 = acc0 + rows_v[row0 + j, sl] * ws[j]
            acc1 = acc1 + rows_v[row0 + j + 1, sl] * ws[j + 1]
          out_v[e, sl] = acc0 + acc1
        return carry

      lax.fori_loop(0, epw, elem, 0)

    def store(out_v, out_h):
      pltpu.make_async_copy(out_v, out_h.at[pl.ds(base_e, epw)],
                            sem_s).start()

    tables = (pc_h, pd_h, gene_h)
    outs_v = (out_c, out_d, out_g)
    outs_h = (sums_c_h, sums_d_h, sums_g_h)

    fire_gathers(0, tables[0], 0)
    fire_gathers(1, tables[1], 1)
    pltpu.make_async_copy(
        pd_h.at[idx_v.at[pl.ds(3 * ipw, epw)]], self_v, sem_f).start()
    for t in range(3):
      q = t % 2
      drain_gathers(q, tables[t])
      accum_type(t, q, outs_v[t])
      if t + 2 < 3:
        fire_gathers(t + 2, tables[t + 2], q)
      store(outs_v[t], outs_h[t])

    pltpu.make_async_copy(pd_h.at[pl.ds(0, epw)], self_v, sem_f).wait()
    store(self_v, self_h)

    pltpu.make_async_copy(sums_c_h.at[pl.ds(0, 4 * epw)],
                          rows_a.at[pl.ds(0, 4 * epw)], sem_s).wait()

  return k(idx_all, w_all, p_cell, p_drug, gene_features)


def _tc_mlp(self_p, sums_c, sums_d, sums_g, cnt_c, cnt_d, cnt_g,
            b_cell, b_drug, W_gene, b_gene, W_l1, b_l1, W_l2, b_l2):
  B = self_p.shape[0]
  f32 = jnp.float32
  inv_m = 1.0 / MAX_NEIGHBORS

  def dot(a, b):
    return lax.dot_general(a, b, (((1,), (0,)), ((), ())),
                           preferred_element_type=f32)

  def body(sp_r, sc_r, sd_r, sg_r, cc_r, cd_r, cg_r,
           bc_r, bd_r, Wg_r, bg_r, Wl1_r, bl1_r, Wl2_r, bl2_r, out_r):
    h = sp_r[...] + bd_r[...]
    agg_c = (sc_r[...] + cc_r[...] * bc_r[...]) * inv_m
    agg_d = (sd_r[...] + cd_r[...] * bd_r[...]) * inv_m
    agg_g = (dot(sg_r[...], Wg_r[...]) + cg_r[...] * bg_r[...]) * inv_m
    for Wl_r, bl_r in ((Wl1_r, bl1_r), (Wl2_r, bl2_r)):
      Wl = Wl_r[...]
      pre = (dot(h, Wl[0:EMBED_D]) + dot(agg_c, Wl[EMBED_D:2 * EMBED_D])
             + dot(agg_d, Wl[2 * EMBED_D:3 * EMBED_D])
             + dot(agg_g, Wl[3 * EMBED_D:4 * EMBED_D]) + bl_r[...])
      h = jnp.maximum(pre, 0.0)
    out_r[...] = h

  E = EMBED_D
  full = lambda shape: pl.BlockSpec(shape, lambda: tuple(0 for _ in shape))
  b2 = lambda v: v.reshape(1, E)
  c2 = lambda v: v.reshape(B, 1)
  return pl.pallas_call(
      body,
      in_specs=[
          full((B, E)), full((B, E)), full((B, E)), full((B, E)),
          full((B, 1)), full((B, 1)), full((B, 1)),
          full((1, E)), full((1, E)),
          full((E, E)), full((1, E)),
          full((4 * E, E)), full((1, E)),
          full((4 * E, E)), full((1, E)),
      ],
      out_specs=full((B, E)),
      out_shape=jax.ShapeDtypeStruct((B, E), jnp.float32),
  )(self_p, sums_c, sums_d, sums_g, c2(cnt_c), c2(cnt_d), c2(cnt_g),
    b2(b_cell), b2(b_drug), W_gene, b2(b_gene),
    W_l1, b2(b_l1), W_l2, b2(b_l2))


def kernel(id_batch, neigh_cell, neigh_drug, neigh_gene,
           cell_features, drug_features, gene_features,
           W_cell, b_cell, W_drug, b_drug, W_gene, b_gene,
           W_l1, b_l1, W_l2, b_l2):
  def prep(neigh):
    mask = neigh != PAD_VALUE
    safe = jnp.where(mask, neigh, 0).astype(jnp.int32)
    idx = safe.reshape(-1)
    w = jnp.pad(mask.astype(jnp.float32),
                ((0, 0), (0, 16 - MAX_NEIGHBORS))).reshape(-1)
    cnt = mask.sum(axis=1).astype(jnp.float32)
    return idx, w, cnt

  idx_c, w_c, cnt_c = prep(neigh_cell)
  idx_d, w_d, cnt_d = prep(neigh_drug)
  idx_g, w_g, cnt_g = prep(neigh_gene)

  B = id_batch.shape[0]
  epw = B // _NW
  ipw = epw * MAX_NEIGHBORS
  r = lambda a, n: a.reshape(_NW, n)
  idx_all = jnp.concatenate(
      [r(idx_c, ipw), r(idx_d, ipw), r(idx_g, ipw),
       r(id_batch.astype(jnp.int32), epw)], axis=1).reshape(-1)
  w_all = jnp.concatenate(
      [r(w_c, epw * 16), r(w_d, epw * 16), r(w_g, epw * 16)],
      axis=1).reshape(-1)

  p_drug = _tc_project(drug_features, W_drug, 400)
  p_cell = _tc_project(cell_features, W_cell, 400)

  sums_c, sums_d, sums_g, self_p = _sc_gather_sums(
      idx_all, w_all, p_cell, p_drug, gene_features, B)

  return _tc_mlp(self_p, sums_c, sums_d, sums_g, cnt_c, cnt_d, cnt_g,
                 b_cell, b_drug, W_gene, b_gene, W_l1, b_l1, W_l2, b_l2)
